# baseline (speedup 1.0000x reference)
import functools

import jax
import jax.numpy as jnp
from jax.experimental import pallas as pl
from jax.experimental.pallas import tpu as pltpu

N = 50000
D = 512
H = 64
B = 1024
BN = 1024
NB = (N + BN - 1) // BN
NPAD = NB * BN


def _pool_kernel(x_ref, ids_ref, w1t_ref, b1_ref, w2_ref, b2_ref,
                 out_ref, acc_ref, den_ref):
    i = pl.program_id(0)

    @pl.when(i == 0)
    def _init():
        acc_ref[...] = jnp.zeros_like(acc_ref)
        den_ref[...] = jnp.zeros_like(den_ref)

    x = x_ref[...]
    h = jnp.tanh(
        jax.lax.dot_general(x, w1t_ref[...], (((1,), (0,)), ((), ())),
                            preferred_element_type=jnp.float32)
        + b1_ref[...])
    s = jnp.sum(h * w2_ref[...], axis=1, keepdims=True) + b2_ref[...]
    e = jnp.exp(s)

    ids = ids_ref[0]
    onehot = (jax.lax.broadcasted_iota(jnp.int32, (B, BN), 0) == ids
              ).astype(jnp.bfloat16)

    ex = (e * x).astype(jnp.bfloat16)
    acc_ref[...] += jax.lax.dot_general(
        onehot, ex, (((1,), (0,)), ((), ())),
        preferred_element_type=jnp.float32)
    den_ref[...] += jax.lax.dot_general(
        onehot.astype(jnp.float32), e, (((1,), (0,)), ((), ())),
        preferred_element_type=jnp.float32)

    @pl.when(i == NB - 1)
    def _finish():
        out_ref[...] = acc_ref[...] / (den_ref[...] + 1e-16)


@jax.jit
def kernel(x, W1, b1, W2, b2, batch):
    ids = batch.astype(jnp.int32)
    x_p = jnp.pad(x, ((0, NPAD - N), (0, 0)))
    ids_p = jnp.pad(ids, (0, NPAD - N), constant_values=B)
    ids3 = ids_p.reshape(NB, 1, BN)

    grid_spec = pltpu.PrefetchScalarGridSpec(
        num_scalar_prefetch=0,
        grid=(NB,),
        in_specs=[
            pl.BlockSpec((BN, D), lambda i: (i, 0)),
            pl.BlockSpec((1, 1, BN), lambda i: (i, 0, 0)),
            pl.BlockSpec((D, H), lambda i: (0, 0)),
            pl.BlockSpec((1, H), lambda i: (0, 0)),
            pl.BlockSpec((1, H), lambda i: (0, 0)),
            pl.BlockSpec((1, 1), lambda i: (0, 0)),
        ],
        out_specs=pl.BlockSpec((B, D), lambda i: (0, 0)),
        scratch_shapes=[
            pltpu.VMEM((B, D), jnp.float32),
            pltpu.VMEM((B, 1), jnp.float32),
        ],
    )
    out = pl.pallas_call(
        _pool_kernel,
        grid_spec=grid_spec,
        out_shape=jax.ShapeDtypeStruct((B, D), jnp.float32),
        compiler_params=pltpu.CompilerParams(
            dimension_semantics=("arbitrary",)),
    )(x_p, ids3, W1.T, b1.reshape(1, H), W2.reshape(1, H),
      b2.reshape(1, 1))
    return out


# speedup vs baseline: 1.5618x; 1.5618x over previous
import jax
import jax.numpy as jnp
from jax.experimental import pallas as pl
from jax.experimental.pallas import tpu as pltpu

N = 50000
D = 512
H = 64
B = 1024
BN = 2048
NB = (N + BN - 1) // BN
DA = D + 128


def _pool_kernel(x_ref, ids_ref, w1t_ref, b1_ref, w2_ref, b2_ref,
                 out_ref, acc_ref):
    i = pl.program_id(0)

    @pl.when(i == 0)
    def _init():
        acc_ref[...] = jnp.zeros_like(acc_ref)

    x = x_ref[...]
    h = jnp.tanh(
        jax.lax.dot_general(x.astype(jnp.bfloat16), w1t_ref[...],
                            (((1,), (0,)), ((), ())),
                            preferred_element_type=jnp.float32)
        + b1_ref[...])
    s = jnp.sum(h * w2_ref[...], axis=1, keepdims=True) + b2_ref[...]
    e = jnp.exp(s)

    valid = (jax.lax.broadcasted_iota(jnp.int32, (BN, 1), 0) + i * BN) < N
    e = jnp.where(valid, e, 0.0)

    ids = ids_ref[0]
    onehot = (jax.lax.broadcasted_iota(jnp.int32, (B, BN), 0) == ids
              ).astype(jnp.bfloat16)

    ex = jnp.where(valid, e * x, 0.0).astype(jnp.bfloat16)
    rhs = jnp.concatenate(
        [ex, jnp.pad(e.astype(jnp.bfloat16), ((0, 0), (0, 127)))], axis=1)
    acc_ref[...] += jax.lax.dot_general(
        onehot, rhs, (((1,), (0,)), ((), ())),
        preferred_element_type=jnp.float32)

    @pl.when(i == NB - 1)
    def _finish():
        out_ref[...] = acc_ref[:, :D] / (acc_ref[:, D:D + 1] + 1e-16)


@jax.jit
def kernel(x, W1, b1, W2, b2, batch):
    ids = batch.astype(jnp.int32)
    ids_p = jnp.pad(ids, (0, NB * BN - N), constant_values=B)
    ids3 = ids_p.reshape(NB, 1, BN)

    grid_spec = pltpu.PrefetchScalarGridSpec(
        num_scalar_prefetch=0,
        grid=(NB,),
        in_specs=[
            pl.BlockSpec((BN, D), lambda i: (i, 0)),
            pl.BlockSpec((1, 1, BN), lambda i: (i, 0, 0)),
            pl.BlockSpec((D, H), lambda i: (0, 0)),
            pl.BlockSpec((1, H), lambda i: (0, 0)),
            pl.BlockSpec((1, H), lambda i: (0, 0)),
            pl.BlockSpec((1, 1), lambda i: (0, 0)),
        ],
        out_specs=pl.BlockSpec((B, D), lambda i: (0, 0)),
        scratch_shapes=[
            pltpu.VMEM((B, DA), jnp.float32),
        ],
    )
    out = pl.pallas_call(
        _pool_kernel,
        grid_spec=grid_spec,
        out_shape=jax.ShapeDtypeStruct((B, D), jnp.float32),
        compiler_params=pltpu.CompilerParams(
            dimension_semantics=("arbitrary",)),
    )(x, ids3, W1.T.astype(jnp.bfloat16), b1.reshape(1, H),
      W2.reshape(1, H), b2.reshape(1, 1))
    return out


# speedup vs baseline: 1.7943x; 1.1489x over previous
import jax
import jax.numpy as jnp
from jax.experimental import pallas as pl
from jax.experimental.pallas import tpu as pltpu

N = 50000
D = 512
H = 64
B = 1024
BN = 1024
NB = (N + BN - 1) // BN
SB = 128
NW = B // SB
ACC_R = B + SB


def _pool_kernel(pre_ref, x_ref, ids_ref, w1t_ref, b1_ref, w2_ref, b2_ref,
                 out_ref, acc_ref, den_ref):
    i = pl.program_id(0)

    @pl.when(i == 0)
    def _init():
        acc_ref[...] = jnp.zeros_like(acc_ref)
        den_ref[...] = jnp.zeros_like(den_ref)

    x = x_ref[...]
    h = jnp.tanh(
        jax.lax.dot_general(x.astype(jnp.bfloat16), w1t_ref[...],
                            (((1,), (0,)), ((), ())),
                            preferred_element_type=jnp.float32)
        + b1_ref[...])
    s = jnp.sum(h * w2_ref[...], axis=1, keepdims=True) + b2_ref[...]
    e = jnp.exp(s)

    valid = (jax.lax.broadcasted_iota(jnp.int32, (BN, 1), 0) + i * BN) < N
    e = jnp.where(valid, e, 0.0)
    ex = jnp.where(valid, e * x, 0.0).astype(jnp.bfloat16)
    e_row = jnp.transpose(e)

    ids = ids_ref[0]
    base = pre_ref[0, i]
    base_al = (base // 8) * 8
    max_rel = pre_ref[1, i] - base_al

    def window(w):
        rel = ids - (base_al + w * SB)
        match = jax.lax.broadcasted_iota(jnp.int32, (SB, BN), 0) == rel
        onehot = match.astype(jnp.bfloat16)
        partial = jax.lax.dot_general(
            onehot, ex, (((1,), (0,)), ((), ())),
            preferred_element_type=jnp.float32)
        dsum = jnp.sum(jnp.where(match, e_row, 0.0), axis=1,
                       keepdims=True)
        start = base_al + w * SB
        acc_ref[pl.ds(start, SB), :] += partial
        den_ref[pl.ds(start, SB), :] += dsum

    window(0)
    for w in range(1, NW):
        @pl.when(max_rel >= w * SB)
        def _w(w=w):
            window(w)

    @pl.when(i == NB - 1)
    def _finish():
        out_ref[...] = acc_ref[:B, :] / (den_ref[:B, :] + 1e-16)


@jax.jit
def kernel(x, W1, b1, W2, b2, batch):
    ids = batch.astype(jnp.int32)
    ids_p = jnp.pad(ids, (0, NB * BN - N), constant_values=B)
    ids3 = ids_p.reshape(NB, 1, BN)
    pre = jnp.stack([ids_p[::BN], ids_p[BN - 1::BN]])

    grid_spec = pltpu.PrefetchScalarGridSpec(
        num_scalar_prefetch=1,
        grid=(NB,),
        in_specs=[
            pl.BlockSpec((BN, D), lambda i, pre: (i, 0)),
            pl.BlockSpec((1, 1, BN), lambda i, pre: (i, 0, 0)),
            pl.BlockSpec((D, H), lambda i, pre: (0, 0)),
            pl.BlockSpec((1, H), lambda i, pre: (0, 0)),
            pl.BlockSpec((1, H), lambda i, pre: (0, 0)),
            pl.BlockSpec((1, 1), lambda i, pre: (0, 0)),
        ],
        out_specs=pl.BlockSpec((B, D), lambda i, pre: (0, 0)),
        scratch_shapes=[
            pltpu.VMEM((ACC_R, D), jnp.float32),
            pltpu.VMEM((ACC_R, 1), jnp.float32),
        ],
    )
    out = pl.pallas_call(
        _pool_kernel,
        grid_spec=grid_spec,
        out_shape=jax.ShapeDtypeStruct((B, D), jnp.float32),
        compiler_params=pltpu.CompilerParams(
            dimension_semantics=("arbitrary",)),
    )(pre, x, ids3, W1.T.astype(jnp.bfloat16), b1.reshape(1, H),
      W2.reshape(1, H), b2.reshape(1, 1))
    return out


# speedup vs baseline: 3.4095x; 1.9002x over previous
import jax
import jax.numpy as jnp
from jax.experimental import pallas as pl
from jax.experimental.pallas import tpu as pltpu

N = 50000
D = 512
H = 64
B = 1024
BN = 2048
NB = (N + BN - 1) // BN
SB = 128
NW = B // SB
ACC_R = B + SB


def _pool_kernel(pre_ref, x_ref, ids_ref, w1t_ref, b1_ref, w2_ref, b2_ref,
                 out_ref, acc_ref, den_ref):
    i = pl.program_id(0)

    @pl.when(i == 0)
    def _init():
        acc_ref[...] = jnp.zeros_like(acc_ref)
        den_ref[...] = jnp.zeros_like(den_ref)

    x = x_ref[...]
    h = jnp.tanh(
        jax.lax.dot_general(x.astype(jnp.bfloat16), w1t_ref[...],
                            (((1,), (0,)), ((), ())),
                            preferred_element_type=jnp.float32)
        + b1_ref[...])
    s = jnp.sum(h * w2_ref[...], axis=1, keepdims=True) + b2_ref[...]
    e = jnp.exp(s)

    valid = (jax.lax.broadcasted_iota(jnp.int32, (BN, 1), 0) + i * BN) < N
    e = jnp.where(valid, e, 0.0)
    e_row = jnp.transpose(e)

    xb = jnp.where(valid, x, 0.0).astype(jnp.bfloat16)

    ids = ids_ref[0]
    base = pre_ref[0, i]
    base_al = (base // 8) * 8
    max_rel = pre_ref[1, i] - base_al

    def window(w):
        rel = ids - (base_al + w * SB)
        match = jax.lax.broadcasted_iota(jnp.int32, (SB, BN), 0) == rel
        wmatch = jnp.where(match, e_row, 0.0)
        partial = jax.lax.dot_general(
            wmatch.astype(jnp.bfloat16), xb, (((1,), (0,)), ((), ())),
            preferred_element_type=jnp.float32)
        dsum = jnp.sum(wmatch, axis=1, keepdims=True)
        start = base_al + w * SB
        acc_ref[pl.ds(start, SB), :] += partial
        den_ref[pl.ds(start, SB), :] += dsum

    window(0)
    for w in range(1, NW):
        @pl.when(max_rel >= w * SB)
        def _w(w=w):
            window(w)

    @pl.when(i == NB - 1)
    def _finish():
        out_ref[...] = acc_ref[:B, :] / (den_ref[:B, :] + 1e-16)


@jax.jit
def kernel(x, W1, b1, W2, b2, batch):
    ids = batch.astype(jnp.int32)
    ids_p = jnp.pad(ids, (0, NB * BN - N), constant_values=B)
    ids3 = ids_p.reshape(NB, 1, BN)
    pre = jnp.stack([ids_p[::BN], ids_p[BN - 1::BN]])

    grid_spec = pltpu.PrefetchScalarGridSpec(
        num_scalar_prefetch=1,
        grid=(NB,),
        in_specs=[
            pl.BlockSpec((BN, D), lambda i, pre: (i, 0)),
            pl.BlockSpec((1, 1, BN), lambda i, pre: (i, 0, 0)),
            pl.BlockSpec((D, H), lambda i, pre: (0, 0)),
            pl.BlockSpec((1, H), lambda i, pre: (0, 0)),
            pl.BlockSpec((1, H), lambda i, pre: (0, 0)),
            pl.BlockSpec((1, 1), lambda i, pre: (0, 0)),
        ],
        out_specs=pl.BlockSpec((B, D), lambda i, pre: (0, 0)),
        scratch_shapes=[
            pltpu.VMEM((ACC_R, D), jnp.float32),
            pltpu.VMEM((ACC_R, 1), jnp.float32),
        ],
    )
    out = pl.pallas_call(
        _pool_kernel,
        grid_spec=grid_spec,
        out_shape=jax.ShapeDtypeStruct((B, D), jnp.float32),
        compiler_params=pltpu.CompilerParams(
            dimension_semantics=("arbitrary",)),
    )(pre, x, ids3, W1.T.astype(jnp.bfloat16), b1.reshape(1, H),
      W2.reshape(1, H), b2.reshape(1, 1))
    return out
